# baseline (speedup 1.0000x reference)
import jax
import jax.numpy as jnp
from jax import lax
from jax.experimental import pallas as pl
from jax.experimental.pallas import tpu as pltpu
from jax.experimental.pallas import tpu_sc as plsc

_NC = 2
_NS = 16
_NW = _NC * _NS

_B = 16384
_S = 50
_D = 32
_CHUNK = 128
_GROUPS = (_B * _S) // (_NW * _CHUNK)
_G = 4
_ROWS = _G * _CHUNK
_NR = _GROUPS // _G
_BBLK = _B // _CHUNK


def _body(table_hbm, idx_hbm, a_hbm, idx_v, rows0, rows1, ob0, ob1,
          sg0, sg1, sw0, sw1):
    w = lax.axis_index("s") * _NC + lax.axis_index("c")
    pltpu.sync_copy(idx_hbm.at[w], idx_v)

    iota16 = lax.iota(jnp.int32, 16)
    cdiag = [jnp.bitwise_and(iota16 + g, 15) for g in range(16)]
    rowc = [(cdiag[g] // 8) * 4 for g in range(16)]
    colc = [lax.rem(cdiag[g], 8) * _CHUNK + iota16 for g in range(16)]

    def fire_g(r, rows, sem):
        for c4 in range(_G):
            pltpu.async_copy(
                table_hbm.at[idx_v.at[r * _G + c4]],
                rows.at[pl.ds(c4 * _CHUNK, _CHUNK)],
                sem,
            )

    def drain_g(rows, sem):
        pltpu.make_async_copy(table_hbm.at[pl.ds(0, _ROWS)], rows, sem).wait()

    def fire_w(r, ob, sem):
        g0 = w * _GROUPS + r * _G
        s_ = g0 // _BBLK
        b0 = g0 % _BBLK
        for db in range(4):
            pltpu.async_copy(
                ob.at[pl.ds(db * _G, _G)], a_hbm.at[s_, db, pl.ds(b0, _G)], sem
            )

    def drain_w(ob, sem):
        pltpu.make_async_copy(a_hbm.at[0, 0, pl.ds(0, 16)], ob, sem).wait()

    def transpose_round(rows, ob):
        @plsc.parallel_loop(0, 8, unroll=4)
        def _(mb):
            mb16 = mb * 16
            for j in range(_G):
                rvec = iota16 + j * _CHUNK + mb16
                for h in range(2):
                    for g in range(16):
                        cvec = cdiag[g] + 16 * h if h else cdiag[g]
                        val = plsc.load_gather(rows, [rvec, cvec])
                        rv2 = rowc[g] + (8 * h + j)
                        cv2 = colc[g] + mb16
                        plsc.store_scatter(ob, [rv2, cv2], val)

    fire_g(0, rows0, sg0)
    fire_g(1, rows1, sg1)

    def outer(k, carry):
        for p, rows, ob, sg, sw in ((0, rows0, ob0, sg0, sw0),
                                    (1, rows1, ob1, sg1, sw1)):
            r = 2 * k + p

            @pl.when(k >= 1)
            def _():
                drain_w(ob, sw)

            drain_g(rows, sg)
            transpose_round(rows, ob)

            @pl.when(r + 2 < _NR)
            def _():
                fire_g(r + 2, rows, sg)

            fire_w(r, ob, sw)
        return carry

    lax.fori_loop(0, _NR // 2, outer, 0)
    drain_w(ob0, sw0)
    drain_w(ob1, sw1)


@jax.jit
def _lookup(table, idx):
    mesh = plsc.VectorSubcoreMesh(core_axis_name="c", subcore_axis_name="s")
    f = pl.kernel(
        _body,
        out_type=jax.ShapeDtypeStruct((_S, 4, _BBLK, 8 * _CHUNK), jnp.float32),
        mesh=mesh,
        scratch_types=[
            pltpu.VMEM((_GROUPS, _CHUNK), jnp.int32),
            pltpu.VMEM((_ROWS, _D), jnp.float32),
            pltpu.VMEM((_ROWS, _D), jnp.float32),
            pltpu.VMEM((4 * _G, 8 * _CHUNK), jnp.float32),
            pltpu.VMEM((4 * _G, 8 * _CHUNK), jnp.float32),
            pltpu.SemaphoreType.DMA,
            pltpu.SemaphoreType.DMA,
            pltpu.SemaphoreType.DMA,
            pltpu.SemaphoreType.DMA,
        ],
        compiler_params=pltpu.CompilerParams(
            use_tc_tiling_on_sc=False, needs_layout_passes=False
        ),
    )
    return f(table, idx)


def kernel(x, table):
    idx = x.T.reshape(_NW, _GROUPS, _CHUNK).astype(jnp.int32)
    a = _lookup(table, idx)
    a = a.reshape(_S, 4, _BBLK, 8, _CHUNK)
    return a.transpose(2, 4, 0, 1, 3).reshape(_B, _S, _D)


# speedup vs baseline: 1.1497x; 1.1497x over previous
import jax
import jax.numpy as jnp
from jax import lax
from jax.experimental import pallas as pl
from jax.experimental.pallas import tpu as pltpu
from jax.experimental.pallas import tpu_sc as plsc

_NC = 2
_NS = 16
_NW = _NC * _NS

_B = 16384
_S = 50
_D = 32
_CHUNK = 128
_GROUPS = (_B * _S) // (_NW * _CHUNK)
_G = 4
_ROWS = _G * _CHUNK
_NR = _GROUPS // _G
_BBLK = _B // _CHUNK


def _body(table_hbm, idx_hbm, a_hbm, idx_v, rows0, rows1, ob0, ob1,
          sg0, sg1, sw0, sw1):
    w = lax.axis_index("s") * _NC + lax.axis_index("c")
    pltpu.sync_copy(idx_hbm.at[w], idx_v)

    iota16 = lax.iota(jnp.int32, 16)
    cdiag = [jnp.bitwise_and(iota16 + g, 15) for g in range(16)]
    rowc = [(cdiag[g] // 8) * 4 for g in range(16)]
    colc = [lax.rem(cdiag[g], 8) * _CHUNK + iota16 for g in range(16)]

    def fire_g(r, rows, sem):
        for c4 in range(_G):
            pltpu.async_copy(
                table_hbm.at[idx_v.at[r * _G + c4]],
                rows.at[pl.ds(c4 * _CHUNK, _CHUNK)],
                sem,
            )

    def drain_g(rows, sem):
        pltpu.make_async_copy(table_hbm.at[pl.ds(0, _ROWS)], rows, sem).wait()

    def fire_w(r, ob, sem):
        g0 = w * _GROUPS + r * _G
        s_ = g0 // _BBLK
        b0 = g0 % _BBLK
        for db in range(4):
            pltpu.async_copy(
                ob.at[pl.ds(db * _G, _G)], a_hbm.at[s_, db, pl.ds(b0, _G)], sem
            )

    def drain_w(ob, sem):
        pltpu.make_async_copy(a_hbm.at[0, 0, pl.ds(0, 16)], ob, sem).wait()

    def transpose_round(rows, ob):
        @plsc.parallel_loop(0, 8, unroll=2)
        def _(mb):
            mb16 = mb * 16
            for j in range(_G):
                rvec = iota16 + j * _CHUNK + mb16
                for h in range(2):
                    for g in range(16):
                        cvec = cdiag[g] + 16 * h if h else cdiag[g]
                        val = plsc.load_gather(rows, [rvec, cvec])
                        rv2 = rowc[g] + (8 * h + j)
                        cv2 = colc[g] + mb16
                        plsc.store_scatter(ob, [rv2, cv2], val)

    fire_g(0, rows0, sg0)
    fire_g(1, rows1, sg1)

    def outer(k, carry):
        for p, rows, ob, sg, sw in ((0, rows0, ob0, sg0, sw0),
                                    (1, rows1, ob1, sg1, sw1)):
            r = 2 * k + p

            @pl.when(k >= 1)
            def _():
                drain_w(ob, sw)

            drain_g(rows, sg)
            transpose_round(rows, ob)

            @pl.when(r + 2 < _NR)
            def _():
                fire_g(r + 2, rows, sg)

            fire_w(r, ob, sw)
        return carry

    lax.fori_loop(0, _NR // 2, outer, 0)
    drain_w(ob0, sw0)
    drain_w(ob1, sw1)


@jax.jit
def _lookup(table, idx):
    mesh = plsc.VectorSubcoreMesh(core_axis_name="c", subcore_axis_name="s")
    f = pl.kernel(
        _body,
        out_type=jax.ShapeDtypeStruct((_S, 4, _BBLK, 8 * _CHUNK), jnp.float32),
        mesh=mesh,
        scratch_types=[
            pltpu.VMEM((_GROUPS, _CHUNK), jnp.int32),
            pltpu.VMEM((_ROWS, _D), jnp.float32),
            pltpu.VMEM((_ROWS, _D), jnp.float32),
            pltpu.VMEM((4 * _G, 8 * _CHUNK), jnp.float32),
            pltpu.VMEM((4 * _G, 8 * _CHUNK), jnp.float32),
            pltpu.SemaphoreType.DMA,
            pltpu.SemaphoreType.DMA,
            pltpu.SemaphoreType.DMA,
            pltpu.SemaphoreType.DMA,
        ],
        compiler_params=pltpu.CompilerParams(
            use_tc_tiling_on_sc=False, needs_layout_passes=False
        ),
    )
    return f(table, idx)


def kernel(x, table):
    idx = x.T.reshape(_NW, _GROUPS, _CHUNK).astype(jnp.int32)
    a = _lookup(table, idx)
    a = a.reshape(_S, 4, _BBLK, 8, _CHUNK)
    return a.transpose(2, 4, 0, 1, 3).reshape(_B, _S, _D)
